# baseline (speedup 1.0000x reference)
import functools

import jax
import jax.numpy as jnp
from jax import lax
from jax.experimental import pallas as pl
from jax.experimental.pallas import tpu as pltpu
from jax.experimental.pallas import tpu_sc as plsc

B = 16384
D = 64
NW = 32
BPW = B // NW
GRP = 16
NGRP = BPW // GRP
L = 16

NU = 1000000
NR = 1000
NR_PAD = 1024
BLK = 128
NBLK = NU // BLK
NBLK_PAD = NBLK + 1
RBLK = NR_PAD // BLK
TAIL = NU - NBLK * BLK


def _detile_kernel(user_t, item_t, rel_t, utail_t, ittail_t, us, its, rs,
                  sem):
    wid = lax.axis_index("s") * 2 + lax.axis_index("c")
    LAG = 6

    def blkcopy(src, dst, blk):
        col = pl.multiple_of(blk * BLK, BLK)
        row = pl.multiple_of(blk * D, 8)
        return pltpu.async_copy(src.at[:, pl.ds(col, BLK)],
                                dst.at[pl.ds(row, D), :], sem)

    def drain_one():
        pltpu.make_async_copy(user_t.at[:, pl.ds(0, BLK)],
                              us.at[pl.ds(0, D), :], sem).wait()

    def body(j, _):
        blk = jnp.minimum(wid + j * NW, NBLK - 1)
        blkcopy(user_t, us, blk)
        blkcopy(item_t, its, blk)
        @pl.when(j >= LAG)
        def _():
            drain_one()
            drain_one()
        return 0

    niter = NBLK // NW + 1
    lax.fori_loop(0, niter, body, 0)
    for _ in range(2 * min(LAG, niter)):
        drain_one()

    @pl.when(wid == 0)
    def _():
        pltpu.sync_copy(utail_t, us.at[pl.ds(NBLK * D, D), :])
    @pl.when(wid == 1)
    def _():
        pltpu.sync_copy(ittail_t, its.at[pl.ds(NBLK * D, D), :])
    for rb in range(RBLK):
        @pl.when(wid == 2 + rb)
        def _(rb=rb):
            pltpu.sync_copy(rel_t.at[:, pl.ds(rb * BLK, BLK)],
                            rs.at[pl.ds(rb * D, D), :])


def _score_kernel(us, its, rs, hidx_hbm, ridx_hbm, tidx_hbm,
                  out_hbm, hidx_v, ridx_v, tidx_v, h_v, r_v, t_v, s_v, sem):
    wid = lax.axis_index("s") * 2 + lax.axis_index("c")
    base = wid * BPW

    pltpu.sync_copy(hidx_hbm.at[pl.ds(base, BPW)], hidx_v)
    pltpu.sync_copy(ridx_hbm.at[pl.ds(base, BPW)], ridx_v)
    pltpu.sync_copy(tidx_hbm.at[pl.ds(base, BPW)], tidx_v)

    lanes = lax.iota(jnp.int32, L)
    lane0 = lanes == 0
    perms = [lanes ^ s for s in (8, 4, 2, 1)]

    def sub16(tab, idx):
        row = pl.multiple_of((idx >> 7) * D, 8)
        co = pl.multiple_of((idx & 127) >> 4 << 4, 16)
        return tab.at[pl.ds(row, D), pl.ds(co, L)]

    def group(g, _):
        sl16 = pl.ds(g * GRP, GRP)
        hsc = [hidx_v[sl16][j] for j in range(GRP)]
        rsc = [ridx_v[sl16][j] for j in range(GRP)]
        tsc = [tidx_v[sl16][j] for j in range(GRP)]
        for j in range(GRP):
            dst = pl.ds(j * D, D)
            pltpu.async_copy(sub16(us, hsc[j]), h_v.at[dst, :], sem)
            pltpu.async_copy(sub16(rs, rsc[j]), r_v.at[dst, :], sem)
            pltpu.async_copy(sub16(its, tsc[j]), t_v.at[dst, :], sem)
        for j in range(GRP):
            dst = pl.ds(j * D, D)
            pltpu.make_async_copy(sub16(us, 0), h_v.at[dst, :], sem).wait()
            pltpu.make_async_copy(sub16(rs, 0), r_v.at[dst, :], sem).wait()
            pltpu.make_async_copy(sub16(its, 0), t_v.at[dst, :], sem).wait()
        for j in range(GRP):
            i = g * GRP + j
            hl = jnp.broadcast_to(hsc[j] & 15, (L,))
            rl = jnp.broadcast_to(rsc[j] & 15, (L,))
            tl = jnp.broadcast_to(tsc[j] & 15, (L,))
            acc = jnp.zeros((L,), jnp.float32)
            for c in range(D // L):
                rows = j * D + c * L + lanes
                hv = plsc.load_gather(h_v, [rows, hl])
                rv = plsc.load_gather(r_v, [rows, rl])
                tv = plsc.load_gather(t_v, [rows, tl])
                acc = acc + jnp.abs(hv + rv - tv)
            for p in perms:
                acc = acc + acc.at[p].get(mode="promise_in_bounds", unique_indices=True)
            plsc.store_scatter(s_v, [jnp.broadcast_to(i, (L,))], -acc, mask=lane0)
        return 0

    lax.fori_loop(0, NGRP, group, 0)

    pltpu.sync_copy(s_v, out_hbm.at[pl.ds(base, BPW)])


@jax.jit
def _score(user_table, item_table, rel_table, head_idx, relation_idx, tail_idx):
    mesh = plsc.VectorSubcoreMesh(core_axis_name="c", subcore_axis_name="s")
    params = pltpu.CompilerParams(needs_layout_passes=False)
    params_sc = pltpu.CompilerParams(
        needs_layout_passes=False, use_tc_tiling_on_sc=False)

    detile = functools.partial(
        pl.kernel,
        mesh=mesh,
        compiler_params=params,
        out_type=(
            jax.ShapeDtypeStruct((NBLK_PAD * D, BLK), jnp.float32),
            jax.ShapeDtypeStruct((NBLK_PAD * D, BLK), jnp.float32),
            jax.ShapeDtypeStruct((RBLK * D, BLK), jnp.float32),
        ),
        scratch_types=[pltpu.SemaphoreType.DMA],
    )(_detile_kernel)

    score = functools.partial(
        pl.kernel,
        mesh=mesh,
        compiler_params=params_sc,
        out_type=jax.ShapeDtypeStruct((B,), jnp.float32),
        scratch_types=[
            pltpu.VMEM((BPW,), jnp.int32),
            pltpu.VMEM((BPW,), jnp.int32),
            pltpu.VMEM((BPW,), jnp.int32),
            pltpu.VMEM((GRP * D, L), jnp.float32),
            pltpu.VMEM((GRP * D, L), jnp.float32),
            pltpu.VMEM((GRP * D, L), jnp.float32),
            pltpu.VMEM((BPW,), jnp.float32),
            pltpu.SemaphoreType.DMA,
        ],
    )(_score_kernel)

    rel_pad = jnp.pad(rel_table, ((0, NR_PAD - NR), (0, 0)))
    utail = jnp.pad(user_table[NBLK * BLK:], ((0, BLK - TAIL), (0, 0)))
    ittail = jnp.pad(item_table[NBLK * BLK:], ((0, BLK - TAIL), (0, 0)))
    us, its, rs = detile(user_table.T, item_table.T, rel_pad.T,
                         utail.T, ittail.T)
    return score(us, its, rs, head_idx, relation_idx, tail_idx)


def kernel(user_table, item_table, rel_table, head_idx, relation_idx, tail_idx):
    return _score(user_table, item_table, rel_table,
                  head_idx.astype(jnp.int32),
                  relation_idx.astype(jnp.int32),
                  tail_idx.astype(jnp.int32))


# speedup vs baseline: 28.3548x; 28.3548x over previous
import functools

import jax
import jax.numpy as jnp
from jax import lax
from jax.experimental import pallas as pl
from jax.experimental.pallas import tpu as pltpu
from jax.experimental.pallas import tpu_sc as plsc

B = 16384
D = 64
NW = 32
BPW = B // NW
GRP = 16
NGRP = BPW // GRP
L = 16

NU = 1000000
NR = 1000
NR_PAD = 1024
BLK = 128
NBLK = NU // BLK
NBLK_PAD = NBLK + 1
RBLK = NR_PAD // BLK
TAIL = NU - NBLK * BLK


def _detile_kernel(user_t, item_t, rel_t, utail_t, ittail_t, us, its, rs,
                  buf, rsem, wsem):
    wid = lax.axis_index("s") * 2 + lax.axis_index("c")
    CB = 4
    CW = CB * BLK
    NCHK = NBLK // CB
    PW = (NCHK + NW - 1) // NW

    for src, dst in ((user_t, us), (item_t, its)):
        def read(j):
            c = jnp.minimum(wid + j * NW, NCHK - 1)
            p = (j & 1) * D
            pltpu.async_copy(src.at[:, pl.ds(pl.multiple_of(c * CW, CW), CW)],
                             buf.at[pl.ds(pl.multiple_of(p, D), D), :], rsem)

        def wait_read():
            pltpu.make_async_copy(src.at[:, pl.ds(0, CW)],
                                  buf.at[pl.ds(0, D), :], rsem).wait()

        def wait_write():
            pltpu.make_async_copy(buf.at[pl.ds(0, D), pl.ds(0, BLK)],
                                  dst.at[pl.ds(0, D), :], wsem).wait()

        read(0)

        def body(j, _):
            c = jnp.minimum(wid + j * NW, NCHK - 1)
            p = (j & 1) * D
            wait_read()
            @pl.when(j + 1 < PW)
            def _():
                read(j + 1)
            for b in range(CB):
                row = pl.multiple_of((c * CB + b) * D, D)
                pltpu.async_copy(
                    buf.at[pl.ds(pl.multiple_of(p, D), D),
                           pl.ds(b * BLK, BLK)],
                    dst.at[pl.ds(row, D), :], wsem)
            for _b in range(CB):
                wait_write()
            return 0

        lax.fori_loop(0, PW, body, 0)

    @pl.when(wid == 0)
    def _():
        pltpu.sync_copy(utail_t, us.at[pl.ds(NBLK * D, D), :])
    @pl.when(wid == 1)
    def _():
        pltpu.sync_copy(ittail_t, its.at[pl.ds(NBLK * D, D), :])
    for rb in range(RBLK):
        @pl.when(wid == 2 + rb)
        def _(rb=rb):
            pltpu.sync_copy(rel_t.at[:, pl.ds(rb * BLK, BLK)],
                            rs.at[pl.ds(rb * D, D), :])


def _score_kernel(us, its, rs, hidx_hbm, ridx_hbm, tidx_hbm,
                  out_hbm, hidx_v, ridx_v, tidx_v, h_v, r_v, t_v, s_v, sem):
    wid = lax.axis_index("s") * 2 + lax.axis_index("c")
    base = wid * BPW

    pltpu.sync_copy(hidx_hbm.at[pl.ds(base, BPW)], hidx_v)
    pltpu.sync_copy(ridx_hbm.at[pl.ds(base, BPW)], ridx_v)
    pltpu.sync_copy(tidx_hbm.at[pl.ds(base, BPW)], tidx_v)

    lanes = lax.iota(jnp.int32, L)
    lane0 = lanes == 0
    perms = [lanes ^ s for s in (8, 4, 2, 1)]

    def sub16(tab, idx):
        row = pl.multiple_of((idx >> 7) * D, 8)
        co = pl.multiple_of((idx & 127) >> 4 << 4, 16)
        return tab.at[pl.ds(row, D), pl.ds(co, L)]

    def group(g, _):
        sl16 = pl.ds(g * GRP, GRP)
        hsc = [hidx_v[sl16][j] for j in range(GRP)]
        rsc = [ridx_v[sl16][j] for j in range(GRP)]
        tsc = [tidx_v[sl16][j] for j in range(GRP)]
        for j in range(GRP):
            dst = pl.ds(j * D, D)
            pltpu.async_copy(sub16(us, hsc[j]), h_v.at[dst, :], sem)
            pltpu.async_copy(sub16(rs, rsc[j]), r_v.at[dst, :], sem)
            pltpu.async_copy(sub16(its, tsc[j]), t_v.at[dst, :], sem)
        for j in range(GRP):
            dst = pl.ds(j * D, D)
            pltpu.make_async_copy(sub16(us, 0), h_v.at[dst, :], sem).wait()
            pltpu.make_async_copy(sub16(rs, 0), r_v.at[dst, :], sem).wait()
            pltpu.make_async_copy(sub16(its, 0), t_v.at[dst, :], sem).wait()
        for j in range(GRP):
            i = g * GRP + j
            hl = jnp.broadcast_to(hsc[j] & 15, (L,))
            rl = jnp.broadcast_to(rsc[j] & 15, (L,))
            tl = jnp.broadcast_to(tsc[j] & 15, (L,))
            acc = jnp.zeros((L,), jnp.float32)
            for c in range(D // L):
                rows = j * D + c * L + lanes
                hv = plsc.load_gather(h_v, [rows, hl])
                rv = plsc.load_gather(r_v, [rows, rl])
                tv = plsc.load_gather(t_v, [rows, tl])
                acc = acc + jnp.abs(hv + rv - tv)
            for p in perms:
                acc = acc + acc.at[p].get(mode="promise_in_bounds", unique_indices=True)
            plsc.store_scatter(s_v, [jnp.broadcast_to(i, (L,))], -acc, mask=lane0)
        return 0

    lax.fori_loop(0, NGRP, group, 0)

    pltpu.sync_copy(s_v, out_hbm.at[pl.ds(base, BPW)])


@jax.jit
def _score(user_table, item_table, rel_table, head_idx, relation_idx, tail_idx):
    mesh = plsc.VectorSubcoreMesh(core_axis_name="c", subcore_axis_name="s")
    params = pltpu.CompilerParams(needs_layout_passes=False)
    params_sc = pltpu.CompilerParams(
        needs_layout_passes=False, use_tc_tiling_on_sc=False)

    detile = functools.partial(
        pl.kernel,
        mesh=mesh,
        compiler_params=params,
        out_type=(
            jax.ShapeDtypeStruct((NBLK_PAD * D, BLK), jnp.float32),
            jax.ShapeDtypeStruct((NBLK_PAD * D, BLK), jnp.float32),
            jax.ShapeDtypeStruct((RBLK * D, BLK), jnp.float32),
        ),
        scratch_types=[
            pltpu.VMEM((2 * D, 4 * BLK), jnp.float32),
            pltpu.SemaphoreType.DMA,
            pltpu.SemaphoreType.DMA,
        ],
    )(_detile_kernel)

    score = functools.partial(
        pl.kernel,
        mesh=mesh,
        compiler_params=params_sc,
        out_type=jax.ShapeDtypeStruct((B,), jnp.float32),
        scratch_types=[
            pltpu.VMEM((BPW,), jnp.int32),
            pltpu.VMEM((BPW,), jnp.int32),
            pltpu.VMEM((BPW,), jnp.int32),
            pltpu.VMEM((GRP * D, L), jnp.float32),
            pltpu.VMEM((GRP * D, L), jnp.float32),
            pltpu.VMEM((GRP * D, L), jnp.float32),
            pltpu.VMEM((BPW,), jnp.float32),
            pltpu.SemaphoreType.DMA,
        ],
    )(_score_kernel)

    rel_pad = jnp.pad(rel_table, ((0, NR_PAD - NR), (0, 0)))
    utail = jnp.pad(user_table[NBLK * BLK:], ((0, BLK - TAIL), (0, 0)))
    ittail = jnp.pad(item_table[NBLK * BLK:], ((0, BLK - TAIL), (0, 0)))
    us, its, rs = detile(user_table.T, item_table.T, rel_pad.T,
                         utail.T, ittail.T)
    return score(us, its, rs, head_idx, relation_idx, tail_idx)


def kernel(user_table, item_table, rel_table, head_idx, relation_idx, tail_idx):
    return _score(user_table, item_table, rel_table,
                  head_idx.astype(jnp.int32),
                  relation_idx.astype(jnp.int32),
                  tail_idx.astype(jnp.int32))


# speedup vs baseline: 29.5059x; 1.0406x over previous
import functools

import jax
import jax.numpy as jnp
from jax import lax
from jax.experimental import pallas as pl
from jax.experimental.pallas import tpu as pltpu
from jax.experimental.pallas import tpu_sc as plsc

B = 16384
D = 64
NW = 32
BPW = B // NW
GRP = 16
NGRP = BPW // GRP
L = 16

NU = 1000000
NR = 1000
NR_PAD = 1024
BLK = 128
NBLK = NU // BLK
NBLK_PAD = NBLK + 1
RBLK = NR_PAD // BLK
TAIL = NU - NBLK * BLK


def _detile_kernel(user_t, item_t, rel_t, utail_t, ittail_t, us, its, rs,
                  buf, rsem, wsem):
    wid = lax.axis_index("s") * 2 + lax.axis_index("c")
    CB = 4
    CW = CB * BLK
    NCHK = NBLK // CB
    PW = (NCHK + NW - 1) // NW

    for src, dst in ((user_t, us), (item_t, its)):
        def read(j):
            c = jnp.minimum(wid + j * NW, NCHK - 1)
            p = (j & 1) * D
            pltpu.async_copy(src.at[:, pl.ds(pl.multiple_of(c * CW, CW), CW)],
                             buf.at[pl.ds(pl.multiple_of(p, D), D), :], rsem)

        def wait_read():
            pltpu.make_async_copy(src.at[:, pl.ds(0, CW)],
                                  buf.at[pl.ds(0, D), :], rsem).wait()

        def wait_write():
            pltpu.make_async_copy(buf.at[pl.ds(0, D), pl.ds(0, BLK)],
                                  dst.at[pl.ds(0, D), :], wsem).wait()

        read(0)

        def body(j, _):
            c = jnp.minimum(wid + j * NW, NCHK - 1)
            p = (j & 1) * D
            wait_read()
            @pl.when(j > 0)
            def _():
                for _b in range(CB):
                    wait_write()
            @pl.when(j + 1 < PW)
            def _():
                read(j + 1)
            for b in range(CB):
                row = pl.multiple_of((c * CB + b) * D, D)
                pltpu.async_copy(
                    buf.at[pl.ds(pl.multiple_of(p, D), D),
                           pl.ds(b * BLK, BLK)],
                    dst.at[pl.ds(row, D), :], wsem)
            return 0

        lax.fori_loop(0, PW, body, 0)
        for _b in range(CB):
            wait_write()

    @pl.when(wid == 0)
    def _():
        pltpu.sync_copy(utail_t, us.at[pl.ds(NBLK * D, D), :])
    @pl.when(wid == 1)
    def _():
        pltpu.sync_copy(ittail_t, its.at[pl.ds(NBLK * D, D), :])
    for rb in range(RBLK):
        @pl.when(wid == 2 + rb)
        def _(rb=rb):
            pltpu.sync_copy(rel_t.at[:, pl.ds(rb * BLK, BLK)],
                            rs.at[pl.ds(rb * D, D), :])


def _score_kernel(us, its, rs, hidx_hbm, ridx_hbm, tidx_hbm,
                  out_hbm, hidx_v, ridx_v, tidx_v, h_v, r_v, t_v, s_v, sem):
    wid = lax.axis_index("s") * 2 + lax.axis_index("c")
    base = wid * BPW

    pltpu.sync_copy(hidx_hbm.at[pl.ds(base, BPW)], hidx_v)
    pltpu.sync_copy(ridx_hbm.at[pl.ds(base, BPW)], ridx_v)
    pltpu.sync_copy(tidx_hbm.at[pl.ds(base, BPW)], tidx_v)

    lanes = lax.iota(jnp.int32, L)
    lane0 = lanes == 0
    perms = [lanes ^ s for s in (8, 4, 2, 1)]

    def sub16(tab, idx):
        row = pl.multiple_of((idx >> 7) * D, 8)
        co = pl.multiple_of((idx & 127) >> 4 << 4, 16)
        return tab.at[pl.ds(row, D), pl.ds(co, L)]

    def scalars(g):
        sl16 = pl.ds(g * GRP, GRP)
        return ([hidx_v[sl16][j] for j in range(GRP)],
                [ridx_v[sl16][j] for j in range(GRP)],
                [tidx_v[sl16][j] for j in range(GRP)])

    def fetch(g, par):
        hsc, rsc, tsc = scalars(g)
        for j in range(GRP):
            dst = pl.ds(pl.multiple_of(par * GRP * D + j * D, 8), D)
            pltpu.async_copy(sub16(us, hsc[j]), h_v.at[dst, :], sem)
            pltpu.async_copy(sub16(rs, rsc[j]), r_v.at[dst, :], sem)
            pltpu.async_copy(sub16(its, tsc[j]), t_v.at[dst, :], sem)

    fetch(0, 0)

    def group(g, _):
        par = g & 1
        for j in range(GRP):
            dst = pl.ds(j * D, D)
            pltpu.make_async_copy(sub16(us, 0), h_v.at[dst, :], sem).wait()
            pltpu.make_async_copy(sub16(rs, 0), r_v.at[dst, :], sem).wait()
            pltpu.make_async_copy(sub16(its, 0), t_v.at[dst, :], sem).wait()
        @pl.when(g + 1 < NGRP)
        def _():
            fetch(g + 1, 1 - par)
        hsc, rsc, tsc = scalars(g)
        for j in range(GRP):
            i = g * GRP + j
            hl = jnp.broadcast_to(hsc[j] & 15, (L,))
            rl = jnp.broadcast_to(rsc[j] & 15, (L,))
            tl = jnp.broadcast_to(tsc[j] & 15, (L,))
            acc = jnp.zeros((L,), jnp.float32)
            for c in range(D // L):
                rows = par * GRP * D + j * D + c * L + lanes
                hv = plsc.load_gather(h_v, [rows, hl])
                rv = plsc.load_gather(r_v, [rows, rl])
                tv = plsc.load_gather(t_v, [rows, tl])
                acc = acc + jnp.abs(hv + rv - tv)
            for p in perms:
                acc = acc + acc.at[p].get(mode="promise_in_bounds", unique_indices=True)
            plsc.store_scatter(s_v, [jnp.broadcast_to(i, (L,))], -acc, mask=lane0)
        return 0

    lax.fori_loop(0, NGRP, group, 0)

    pltpu.sync_copy(s_v, out_hbm.at[pl.ds(base, BPW)])


@jax.jit
def _score(user_table, item_table, rel_table, head_idx, relation_idx, tail_idx):
    mesh = plsc.VectorSubcoreMesh(core_axis_name="c", subcore_axis_name="s")
    params = pltpu.CompilerParams(needs_layout_passes=False)
    params_sc = pltpu.CompilerParams(
        needs_layout_passes=False, use_tc_tiling_on_sc=False)

    detile = functools.partial(
        pl.kernel,
        mesh=mesh,
        compiler_params=params,
        out_type=(
            jax.ShapeDtypeStruct((NBLK_PAD * D, BLK), jnp.float32),
            jax.ShapeDtypeStruct((NBLK_PAD * D, BLK), jnp.float32),
            jax.ShapeDtypeStruct((RBLK * D, BLK), jnp.float32),
        ),
        scratch_types=[
            pltpu.VMEM((2 * D, 4 * BLK), jnp.float32),
            pltpu.SemaphoreType.DMA,
            pltpu.SemaphoreType.DMA,
        ],
    )(_detile_kernel)

    score = functools.partial(
        pl.kernel,
        mesh=mesh,
        compiler_params=params_sc,
        out_type=jax.ShapeDtypeStruct((B,), jnp.float32),
        scratch_types=[
            pltpu.VMEM((BPW,), jnp.int32),
            pltpu.VMEM((BPW,), jnp.int32),
            pltpu.VMEM((BPW,), jnp.int32),
            pltpu.VMEM((2 * GRP * D, L), jnp.float32),
            pltpu.VMEM((2 * GRP * D, L), jnp.float32),
            pltpu.VMEM((2 * GRP * D, L), jnp.float32),
            pltpu.VMEM((BPW,), jnp.float32),
            pltpu.SemaphoreType.DMA,
        ],
    )(_score_kernel)

    rel_pad = jnp.pad(rel_table, ((0, NR_PAD - NR), (0, 0)))
    utail = jnp.pad(user_table[NBLK * BLK:], ((0, BLK - TAIL), (0, 0)))
    ittail = jnp.pad(item_table[NBLK * BLK:], ((0, BLK - TAIL), (0, 0)))
    us, its, rs = detile(user_table.T, item_table.T, rel_pad.T,
                         utail.T, ittail.T)
    return score(us, its, rs, head_idx, relation_idx, tail_idx)


def kernel(user_table, item_table, rel_table, head_idx, relation_idx, tail_idx):
    return _score(user_table, item_table, rel_table,
                  head_idx.astype(jnp.int32),
                  relation_idx.astype(jnp.int32),
                  tail_idx.astype(jnp.int32))


# speedup vs baseline: 29.9344x; 1.0145x over previous
import functools

import jax
import jax.numpy as jnp
from jax import lax
from jax.experimental import pallas as pl
from jax.experimental.pallas import tpu as pltpu
from jax.experimental.pallas import tpu_sc as plsc

B = 16384
D = 64
NW = 32
BPW = B // NW
GRP = 16
NGRP = BPW // GRP
L = 16

NU = 1000000
NR = 1000
NR_PAD = 1024
BLK = 128
NBLK = NU // BLK
NBLK_PAD = NBLK + 1
RBLK = NR_PAD // BLK
TAIL = NU - NBLK * BLK


def _detile_kernel(user_t, item_t, rel_t, utail_t, ittail_t, us, its, rs,
                  buf, rsem, wsem):
    wid = lax.axis_index("s") * 2 + lax.axis_index("c")
    CB = 6
    CW = CB * BLK
    NCHK = NBLK // CB
    PW = (NCHK + NW - 1) // NW

    for src, dst in ((user_t, us), (item_t, its)):
        def read(j):
            c = jnp.minimum(wid + j * NW, NCHK - 1)
            p = (j & 1) * D
            pltpu.async_copy(src.at[:, pl.ds(pl.multiple_of(c * CW, CW), CW)],
                             buf.at[pl.ds(pl.multiple_of(p, D), D), :], rsem)

        def wait_read():
            pltpu.make_async_copy(src.at[:, pl.ds(0, CW)],
                                  buf.at[pl.ds(0, D), :], rsem).wait()

        def wait_write():
            pltpu.make_async_copy(buf.at[pl.ds(0, D), pl.ds(0, BLK)],
                                  dst.at[pl.ds(0, D), :], wsem).wait()

        read(0)

        def body(j, _):
            c = jnp.minimum(wid + j * NW, NCHK - 1)
            p = (j & 1) * D
            wait_read()
            @pl.when(j > 0)
            def _():
                for _b in range(CB):
                    wait_write()
            @pl.when(j + 1 < PW)
            def _():
                read(j + 1)
            for b in range(CB):
                row = pl.multiple_of((c * CB + b) * D, D)
                pltpu.async_copy(
                    buf.at[pl.ds(pl.multiple_of(p, D), D),
                           pl.ds(b * BLK, BLK)],
                    dst.at[pl.ds(row, D), :], wsem)
            return 0

        lax.fori_loop(0, PW, body, 0)
        for _b in range(CB):
            wait_write()

    @pl.when(wid == 0)
    def _():
        pltpu.sync_copy(utail_t, us.at[pl.ds(NBLK * D, D), :])
    @pl.when(wid == 1)
    def _():
        pltpu.sync_copy(ittail_t, its.at[pl.ds(NBLK * D, D), :])
    for rb in range(RBLK):
        @pl.when(wid == 2 + rb)
        def _(rb=rb):
            pltpu.sync_copy(rel_t.at[:, pl.ds(rb * BLK, BLK)],
                            rs.at[pl.ds(rb * D, D), :])


def _score_kernel(us, its, rs, hidx_hbm, ridx_hbm, tidx_hbm,
                  out_hbm, hidx_v, ridx_v, tidx_v, h_v, r_v, t_v, s_v, sem):
    wid = lax.axis_index("s") * 2 + lax.axis_index("c")
    base = wid * BPW

    pltpu.sync_copy(hidx_hbm.at[pl.ds(base, BPW)], hidx_v)
    pltpu.sync_copy(ridx_hbm.at[pl.ds(base, BPW)], ridx_v)
    pltpu.sync_copy(tidx_hbm.at[pl.ds(base, BPW)], tidx_v)

    lanes = lax.iota(jnp.int32, L)
    lane0 = lanes == 0
    perms = [lanes ^ s for s in (8, 4, 2, 1)]

    def sub16(tab, idx):
        row = pl.multiple_of((idx >> 7) * D, 8)
        co = pl.multiple_of((idx & 127) >> 4 << 4, 16)
        return tab.at[pl.ds(row, D), pl.ds(co, L)]

    def scalars(g):
        sl16 = pl.ds(g * GRP, GRP)
        return ([hidx_v[sl16][j] for j in range(GRP)],
                [ridx_v[sl16][j] for j in range(GRP)],
                [tidx_v[sl16][j] for j in range(GRP)])

    def fetch(g, par):
        hsc, rsc, tsc = scalars(g)
        for j in range(GRP):
            dst = pl.ds(pl.multiple_of(par * GRP * D + j * D, 8), D)
            pltpu.async_copy(sub16(us, hsc[j]), h_v.at[dst, :], sem)
            pltpu.async_copy(sub16(rs, rsc[j]), r_v.at[dst, :], sem)
            pltpu.async_copy(sub16(its, tsc[j]), t_v.at[dst, :], sem)

    fetch(0, 0)

    def group(g, _):
        par = g & 1
        for j in range(GRP):
            dst = pl.ds(j * D, D)
            pltpu.make_async_copy(sub16(us, 0), h_v.at[dst, :], sem).wait()
            pltpu.make_async_copy(sub16(rs, 0), r_v.at[dst, :], sem).wait()
            pltpu.make_async_copy(sub16(its, 0), t_v.at[dst, :], sem).wait()
        @pl.when(g + 1 < NGRP)
        def _():
            fetch(g + 1, 1 - par)
        hsc, rsc, tsc = scalars(g)
        for j in range(GRP):
            i = g * GRP + j
            hl = jnp.broadcast_to(hsc[j] & 15, (L,))
            rl = jnp.broadcast_to(rsc[j] & 15, (L,))
            tl = jnp.broadcast_to(tsc[j] & 15, (L,))
            acc = jnp.zeros((L,), jnp.float32)
            for c in range(D // L):
                rows = par * GRP * D + j * D + c * L + lanes
                hv = plsc.load_gather(h_v, [rows, hl])
                rv = plsc.load_gather(r_v, [rows, rl])
                tv = plsc.load_gather(t_v, [rows, tl])
                acc = acc + jnp.abs(hv + rv - tv)
            for p in perms:
                acc = acc + acc.at[p].get(mode="promise_in_bounds", unique_indices=True)
            plsc.store_scatter(s_v, [jnp.broadcast_to(i, (L,))], -acc, mask=lane0)
        return 0

    lax.fori_loop(0, NGRP, group, 0)

    pltpu.sync_copy(s_v, out_hbm.at[pl.ds(base, BPW)])


@jax.jit
def _score(user_table, item_table, rel_table, head_idx, relation_idx, tail_idx):
    mesh = plsc.VectorSubcoreMesh(core_axis_name="c", subcore_axis_name="s")
    params = pltpu.CompilerParams(needs_layout_passes=False)
    params_sc = pltpu.CompilerParams(
        needs_layout_passes=False, use_tc_tiling_on_sc=False)

    detile = functools.partial(
        pl.kernel,
        mesh=mesh,
        compiler_params=params,
        out_type=(
            jax.ShapeDtypeStruct((NBLK_PAD * D, BLK), jnp.float32),
            jax.ShapeDtypeStruct((NBLK_PAD * D, BLK), jnp.float32),
            jax.ShapeDtypeStruct((RBLK * D, BLK), jnp.float32),
        ),
        scratch_types=[
            pltpu.VMEM((2 * D, 6 * BLK), jnp.float32),
            pltpu.SemaphoreType.DMA,
            pltpu.SemaphoreType.DMA,
        ],
    )(_detile_kernel)

    score = functools.partial(
        pl.kernel,
        mesh=mesh,
        compiler_params=params_sc,
        out_type=jax.ShapeDtypeStruct((B,), jnp.float32),
        scratch_types=[
            pltpu.VMEM((BPW,), jnp.int32),
            pltpu.VMEM((BPW,), jnp.int32),
            pltpu.VMEM((BPW,), jnp.int32),
            pltpu.VMEM((2 * GRP * D, L), jnp.float32),
            pltpu.VMEM((2 * GRP * D, L), jnp.float32),
            pltpu.VMEM((2 * GRP * D, L), jnp.float32),
            pltpu.VMEM((BPW,), jnp.float32),
            pltpu.SemaphoreType.DMA,
        ],
    )(_score_kernel)

    rel_pad = jnp.pad(rel_table, ((0, NR_PAD - NR), (0, 0)))
    utail = jnp.pad(user_table[NBLK * BLK:], ((0, BLK - TAIL), (0, 0)))
    ittail = jnp.pad(item_table[NBLK * BLK:], ((0, BLK - TAIL), (0, 0)))
    us, its, rs = detile(user_table.T, item_table.T, rel_pad.T,
                         utail.T, ittail.T)
    return score(us, its, rs, head_idx, relation_idx, tail_idx)


def kernel(user_table, item_table, rel_table, head_idx, relation_idx, tail_idx):
    return _score(user_table, item_table, rel_table,
                  head_idx.astype(jnp.int32),
                  relation_idx.astype(jnp.int32),
                  tail_idx.astype(jnp.int32))
